# baseline (speedup 1.0000x reference)
import functools

import jax
import jax.numpy as jnp
from jax import lax
from jax.experimental import pallas as pl
from jax.experimental.pallas import tpu as pltpu
from jax.experimental.pallas import tpu_sc as plsc

B = 16384
D = 64
NUM = 16
EPS = 1e-3

_NC, _NS = 2, 16
_NW = _NC * _NS
_BPW = B // _NW


def _sc_gather(idx_hbm, table_hbm, out_hbm, idx_v, rows_v, sem):
    wid = lax.axis_index("s") * _NC + lax.axis_index("c")
    base = wid * _BPW
    pltpu.sync_copy(idx_hbm.at[pl.ds(base, _BPW)], idx_v)
    pltpu.async_copy(table_hbm.at[idx_v], rows_v, sem).wait()
    pltpu.sync_copy(rows_v, out_hbm.at[pl.ds(base, _BPW)])


@functools.cache
def _gather_call():
    return functools.partial(
        pl.kernel,
        mesh=plsc.VectorSubcoreMesh(core_axis_name="c", subcore_axis_name="s"),
        out_type=jax.ShapeDtypeStruct((B, D), jnp.float32),
        compiler_params=pltpu.CompilerParams(use_tc_tiling_on_sc=False),
        scratch_types=[
            pltpu.VMEM((_BPW,), jnp.int32),
            pltpu.VMEM((_BPW, D), jnp.float32),
            pltpu.SemaphoreType.DMA,
        ],
    )(_sc_gather)


_BLK = 2048


def _mlp_body(emb, num, w1, b1, g1, be1, w2, b2, g2, be2, w3, b3, out):
    s = lax.rsqrt(jnp.float32(1.0 + EPS))
    w1full = w1[...]
    h = jnp.maximum(emb[...] @ w1full[:D] + num[...] @ w1full[D:] + b1[...], 0.0)
    h = h * (s * g1[...]) + be1[...]
    h = jnp.maximum(h @ w2[...] + b2[...], 0.0)
    h = h * (s * g2[...]) + be2[...]
    o = h @ w3[...] + b3[...]
    sq = jnp.sum(o * o, axis=1, keepdims=True)
    out[...] = o * lax.rsqrt(jnp.maximum(sq, 1e-12))


_mlp_call = pl.pallas_call(
    _mlp_body,
    grid=(B // _BLK,),
    in_specs=[
        pl.BlockSpec((_BLK, D), lambda i: (i, 0)),
        pl.BlockSpec((_BLK, NUM), lambda i: (i, 0)),
        pl.BlockSpec((D + NUM, 128), lambda i: (0, 0)),
        pl.BlockSpec((128,), lambda i: (0,)),
        pl.BlockSpec((128,), lambda i: (0,)),
        pl.BlockSpec((128,), lambda i: (0,)),
        pl.BlockSpec((128, 64), lambda i: (0, 0)),
        pl.BlockSpec((64,), lambda i: (0,)),
        pl.BlockSpec((64,), lambda i: (0,)),
        pl.BlockSpec((64,), lambda i: (0,)),
        pl.BlockSpec((64, D), lambda i: (0, 0)),
        pl.BlockSpec((D,), lambda i: (0,)),
    ],
    out_specs=pl.BlockSpec((_BLK, D), lambda i: (i, 0)),
    out_shape=jax.ShapeDtypeStruct((B, D), jnp.float32),
)


def kernel(user_id, user_numerical_features, table, W1, b1, gamma1, beta1,
           W2, b2, gamma2, beta2, W3, b3):
    idx = user_id.astype(jnp.int32)
    emb = _gather_call()(idx, table)
    return _mlp_call(emb, user_numerical_features, W1, b1, gamma1, beta1,
                     W2, b2, gamma2, beta2, W3, b3)


# speedup vs baseline: 1.7542x; 1.7542x over previous
import functools

import jax
import jax.numpy as jnp
from jax import lax
from jax.experimental import pallas as pl
from jax.experimental.pallas import tpu as pltpu
from jax.experimental.pallas import tpu_sc as plsc

B = 16384
V = 1000000
D = 64
NUM = 16
EPS = 1e-3

_NC, _NS = 2, 16
_NW = _NC * _NS
_BPW = B // _NW

_TBLK = 2048
_H = 512000
_HGRID = _H // _TBLK
_CMAX = pl.cdiv(V, _TBLK) - 1


def _xpose_body(tA, tB, out):
    out[...] = jnp.concatenate([tA[...].T, tB[...].T], axis=1)


_xpose_call = pl.pallas_call(
    _xpose_body,
    grid=(_HGRID,),
    in_specs=[
        pl.BlockSpec((D, _TBLK), lambda i: (0, i)),
        pl.BlockSpec((D, _TBLK),
                     lambda i: (0, jnp.minimum(_HGRID + i, _CMAX))),
    ],
    out_specs=pl.BlockSpec((_TBLK, 2 * D), lambda i: (i, 0)),
    out_shape=jax.ShapeDtypeStruct((_H, 2 * D), jnp.float32),
)


def _sc_gather(idx_hbm, packed_hbm, out_hbm, idx_v, idx2_v, rows_v, sem):
    wid = lax.axis_index("s") * _NC + lax.axis_index("c")
    base = wid * _BPW
    pltpu.sync_copy(idx_hbm.at[pl.ds(base, _BPW)], idx_v)

    def tomod(g, carry):
        sl = pl.ds(g * 16, 16)
        v = idx_v[sl]
        idx2_v[sl] = jnp.where(v >= _H, v - _H, v)
        return carry

    lax.fori_loop(0, _BPW // 16, tomod, 0)
    pltpu.async_copy(packed_hbm.at[idx2_v], rows_v, sem).wait()
    pltpu.sync_copy(rows_v, out_hbm.at[pl.ds(base, _BPW)])


@functools.cache
def _gather_call():
    return functools.partial(
        pl.kernel,
        mesh=plsc.VectorSubcoreMesh(core_axis_name="c", subcore_axis_name="s"),
        out_type=jax.ShapeDtypeStruct((B, 2 * D), jnp.float32),
        compiler_params=pltpu.CompilerParams(use_tc_tiling_on_sc=True),
        scratch_types=[
            pltpu.VMEM((_BPW,), jnp.int32),
            pltpu.VMEM((_BPW,), jnp.int32),
            pltpu.VMEM((_BPW, 2 * D), jnp.float32),
            pltpu.SemaphoreType.DMA,
        ],
    )(_sc_gather)


_BLK = 2048


def _mlp_body(emb2, ids, num, w1, b1, g1, be1, w2, b2, g2, be2, w3, b3, outT):
    s = lax.rsqrt(jnp.float32(1.0 + EPS))
    e2 = emb2[...]
    odd = ids[...][:, None] >= _H
    emb = jnp.where(odd, e2[:, D:], e2[:, :D])
    w1full = w1[...]
    h = jnp.maximum(emb @ w1full[:D] + num[...] @ w1full[D:] + b1[...], 0.0)
    h = h * (s * g1[...]) + be1[...]
    h = jnp.maximum(h @ w2[...] + b2[...], 0.0)
    h = h * (s * g2[...]) + be2[...]
    o = h @ w3[...] + b3[...]
    sq = jnp.sum(o * o, axis=1, keepdims=True)
    o = o * lax.rsqrt(jnp.maximum(sq, 1e-12))
    outT[...] = o.T


_mlp_call = pl.pallas_call(
    _mlp_body,
    grid=(B // _BLK,),
    in_specs=[
        pl.BlockSpec((_BLK, 2 * D), lambda i: (i, 0)),
        pl.BlockSpec((_BLK,), lambda i: (i,)),
        pl.BlockSpec((_BLK, NUM), lambda i: (i, 0)),
        pl.BlockSpec((D + NUM, 128), lambda i: (0, 0)),
        pl.BlockSpec((128,), lambda i: (0,)),
        pl.BlockSpec((128,), lambda i: (0,)),
        pl.BlockSpec((128,), lambda i: (0,)),
        pl.BlockSpec((128, 64), lambda i: (0, 0)),
        pl.BlockSpec((64,), lambda i: (0,)),
        pl.BlockSpec((64,), lambda i: (0,)),
        pl.BlockSpec((64,), lambda i: (0,)),
        pl.BlockSpec((64, D), lambda i: (0, 0)),
        pl.BlockSpec((D,), lambda i: (0,)),
    ],
    out_specs=pl.BlockSpec((D, _BLK), lambda i: (0, i)),
    out_shape=jax.ShapeDtypeStruct((D, B), jnp.float32),
)


def kernel(user_id, user_numerical_features, table, W1, b1, gamma1, beta1,
           W2, b2, gamma2, beta2, W3, b3):
    idx = user_id.astype(jnp.int32)
    tableT = table.T
    packed = _xpose_call(tableT, tableT)
    emb2 = _gather_call()(idx, packed)
    outT = _mlp_call(emb2, idx, user_numerical_features, W1, b1, gamma1,
                     beta1, W2, b2, gamma2, beta2, W3, b3)
    return outT.T
